# baseline (speedup 1.0000x reference)
import jax
import jax.numpy as jnp
from jax.experimental import pallas as pl
from jax.experimental.pallas import tpu as pltpu

N = 4096
F = 256
H = 128
TAU = 0.5
BM = 256
NI = N // BM
HC = N // 2


def _cl_gcn_kernel(x1_ref, x2_ref, adj1_ref, adj2_ref, clm_ref,
                   w11_ref, b11_ref, w12_ref, b12_ref,
                   w21_ref, b21_ref, w22_ref, b22_ref,
                   z1_ref, z2_ref, loss_ref,
                   adj_scr, sup1_scr, sup2_scr, s2a_scr, s2b_scr,
                   rs_scr, ws_scr, acc_ref):
    t = pl.program_id(0)

    @pl.when(t < NI)
    def _():
        i = t
        xb1 = x1_ref[...].astype(jnp.bfloat16)
        sup = jnp.dot(xb1, w11_ref[...], preferred_element_type=jnp.float32)
        sup1_scr[pl.ds(i * BM, BM), :] = sup.astype(jnp.bfloat16)
        xb2 = x2_ref[...].astype(jnp.bfloat16)
        sup = jnp.dot(xb2, w21_ref[...], preferred_element_type=jnp.float32)
        sup2_scr[pl.ds(i * BM, BM), :] = sup.astype(jnp.bfloat16)

    @pl.when(jnp.logical_and(t >= NI, t < 2 * NI))
    def _():
        i = t - NI
        ab = adj2_ref[...].astype(jnp.bfloat16)
        adj_scr[pl.ds(i * BM, BM), :] = ab
        acc = jnp.dot(ab, sup2_scr[...], preferred_element_type=jnp.float32)
        h = jnp.maximum(acc + b21_ref[...], 0.0)
        s2 = jnp.dot(h.astype(jnp.bfloat16), w22_ref[...],
                     preferred_element_type=jnp.float32)
        s2b_scr[pl.ds(i * BM, BM), :] = s2.astype(jnp.bfloat16)

    @pl.when(jnp.logical_and(t >= 2 * NI, t < 3 * NI))
    def _():
        i = t - 2 * NI
        a2 = adj_scr[pl.ds(i * BM, BM), :]
        z2 = jnp.dot(a2, s2b_scr[...],
                     preferred_element_type=jnp.float32) + b22_ref[...]
        z2_ref[...] = z2
        r2 = jax.lax.rsqrt(jnp.sum(z2 * z2, axis=1, keepdims=True))
        sup2_scr[pl.ds(i * BM, BM), :H] = (z2 * r2).astype(jnp.bfloat16)
        ab = adj1_ref[...].astype(jnp.bfloat16)
        adj_scr[pl.ds(i * BM, BM), :] = ab
        acc = jnp.dot(ab, sup1_scr[...], preferred_element_type=jnp.float32)
        h = jnp.maximum(acc + b11_ref[...], 0.0)
        s2 = jnp.dot(h.astype(jnp.bfloat16), w12_ref[...],
                     preferred_element_type=jnp.float32)
        s2a_scr[pl.ds(i * BM, BM), :] = s2.astype(jnp.bfloat16)

    @pl.when(t >= 3 * NI)
    def _():
        k = t - 3 * NI
        i = k // 2
        half = k % 2

        @pl.when(half == 0)
        def _():
            a1 = adj_scr[pl.ds(i * BM, BM), :]
            z1 = jnp.dot(a1, s2a_scr[...],
                         preferred_element_type=jnp.float32) + b12_ref[...]
            z1_ref[...] = z1
            r1 = jax.lax.rsqrt(jnp.sum(z1 * z1, axis=1, keepdims=True))
            sup1_scr[pl.ds(i * BM, BM), :H] = (z1 * r1 * (1.0 / TAU)
                                               ).astype(jnp.bfloat16)

        z1s = sup1_scr[pl.ds(i * BM, BM), :H]
        z2s = sup2_scr[pl.ds(half * HC, HC), :H]
        s = jax.lax.dot_general(z1s, z2s, (((1,), (1,)), ((), ())),
                                preferred_element_type=jnp.float32)
        pexp = jnp.exp(s)
        prs = jnp.sum(pexp, axis=1, keepdims=True)
        pws = jnp.sum(pexp * clm_ref[...], axis=1, keepdims=True)

        @pl.when(half == 0)
        def _():
            rs_scr[...] = prs
            ws_scr[...] = pws

        @pl.when(half == 1)
        def _():
            rs = rs_scr[...] + prs
            ws = ws_scr[...] + pws
            part = jnp.sum(jnp.log(rs + 1e-8) - jnp.log(ws))

            @pl.when(i == 0)
            def _():
                acc_ref[0] = 0.0

            acc_ref[0] += part

            @pl.when(i == NI - 1)
            def _():
                loss_ref[...] = jnp.full((1, 1), acc_ref[0] * (1.0 / N),
                                         dtype=jnp.float32)


def _cl_gcn(x1, adj1, x2, adj2, clm,
            W11, b11, W12, b12, W21, b21, W22, b22):
    ni = NI
    z1, z2, loss = pl.pallas_call(
        _cl_gcn_kernel,
        grid=(5 * ni,),
        in_specs=[
            pl.BlockSpec((BM, F),
                         lambda t: (jnp.where(t < ni, t, ni - 1), 0)),
            pl.BlockSpec((BM, F),
                         lambda t: (jnp.where(t < ni, t, ni - 1), 0)),
            pl.BlockSpec((BM, N),
                         lambda t: (jnp.where(t < 2 * ni, 0,
                                              jnp.where(t < 3 * ni, t - 2 * ni,
                                                        ni - 1)), 0)),
            pl.BlockSpec((BM, N),
                         lambda t: (jnp.where(t < ni, 0,
                                              jnp.where(t < 2 * ni, t - ni,
                                                        ni - 1)), 0)),
            pl.BlockSpec((BM, HC),
                         lambda t: (jnp.where(t < 3 * ni, 0,
                                              (t - 3 * ni) // 2),
                                    jnp.where(t < 3 * ni, 0,
                                              (t - 3 * ni) % 2))),
            pl.BlockSpec((F, F), lambda t: (0, 0)),
            pl.BlockSpec((1, F), lambda t: (0, 0)),
            pl.BlockSpec((F, H), lambda t: (0, 0)),
            pl.BlockSpec((1, H), lambda t: (0, 0)),
            pl.BlockSpec((F, F), lambda t: (0, 0)),
            pl.BlockSpec((1, F), lambda t: (0, 0)),
            pl.BlockSpec((F, H), lambda t: (0, 0)),
            pl.BlockSpec((1, H), lambda t: (0, 0)),
        ],
        out_specs=[
            pl.BlockSpec((BM, H),
                         lambda t: (jnp.where(t < 3 * ni, 0,
                                              (t - 3 * ni) // 2), 0)),
            pl.BlockSpec((BM, H),
                         lambda t: (jnp.where(t < 2 * ni, 0,
                                              jnp.where(t < 3 * ni, t - 2 * ni,
                                                        ni - 1)), 0)),
            pl.BlockSpec((1, 1), lambda t: (0, 0)),
        ],
        out_shape=[
            jax.ShapeDtypeStruct((N, H), jnp.float32),
            jax.ShapeDtypeStruct((N, H), jnp.float32),
            jax.ShapeDtypeStruct((1, 1), jnp.float32),
        ],
        scratch_shapes=[
            pltpu.VMEM((N, N), jnp.bfloat16),
            pltpu.VMEM((N, F), jnp.bfloat16),
            pltpu.VMEM((N, F), jnp.bfloat16),
            pltpu.VMEM((N, H), jnp.bfloat16),
            pltpu.VMEM((N, H), jnp.bfloat16),
            pltpu.VMEM((BM, 1), jnp.float32),
            pltpu.VMEM((BM, 1), jnp.float32),
            pltpu.SMEM((1,), jnp.float32),
        ],
        compiler_params=pltpu.CompilerParams(
            vmem_limit_bytes=63 * 1024 * 1024,
        ),
    )(x1, x2, adj1, adj2, clm,
      W11.astype(jnp.bfloat16), b11.reshape(1, F),
      W12.astype(jnp.bfloat16), b12.reshape(1, H),
      W21.astype(jnp.bfloat16), b21.reshape(1, F),
      W22.astype(jnp.bfloat16), b22.reshape(1, H))
    return z1, z2, loss.reshape(())


def kernel(x1, adj1, x2, adj2, clm, W11, b11, W12, b12, W21, b21, W22, b22):
    z1, z2, loss = _cl_gcn(x1, adj1, x2, adj2, clm,
                           W11, b11, W12, b12, W21, b21, W22, b22)
    return (z1, z2, loss)
